# baseline (speedup 1.0000x reference)
import functools

import jax
import jax.numpy as jnp
from jax import lax
from jax.experimental import pallas as pl
from jax.experimental.pallas import tpu as pltpu
from jax.experimental.pallas import tpu_sc as plsc

BATCH = 16384
NUM_COLS = 26
ROWS = 100001
CH = 32

N = BATCH * NUM_COLS
NC = 2
NS = 16
GROUPS_PER_CORE = 2
SEG = 100008
FLAT = NUM_COLS * CH * SEG
BQ = 512
NBQ = BATCH // BQ


RB = 12544
RB_LAST = 12160
SB = 3072
SUBS = (SB, SB, SB, SB, 256)
SUBS_LAST = (SB, SB, SB, 2944)
TAIL0 = 99968
TAILN = ROWS - TAIL0


def _detile_body(tab_hbm, tails_hbm, flat_hbm, row_v, trow_v, sp_blk, sp_tl):
    c0 = lax.axis_index("c")
    s = lax.axis_index("s")
    grp = s // 8
    k = s % 8
    grp = c0 * GROUPS_PER_CORE + grp
    ch8 = grp * 8
    r0 = k * RB

    def col_body(c, _):
        def do_range(subs):
            off = 0
            for ext in subs:
                pltpu.sync_copy(
                    tab_hbm.at[c, pl.ds(ch8, 8), pl.ds(r0 + off, ext)],
                    sp_blk.at[s, :, pl.ds(0, ext)],
                )
                for j in range(8):
                    pltpu.sync_copy(
                        sp_blk.at[s, j, pl.ds(0, ext)],
                        row_v.at[pl.ds(0, ext)],
                    )
                    pltpu.sync_copy(
                        row_v.at[pl.ds(0, ext)],
                        flat_hbm.at[
                            pl.ds((c * CH + ch8 + j) * SEG + r0 + off, ext)
                        ],
                    )
                off += ext

        @pl.when(k < 7)
        def _main():
            do_range(SUBS)

        @pl.when(k == 7)
        def _last():
            do_range(SUBS_LAST)
            pltpu.sync_copy(tails_hbm.at[c, pl.ds(ch8, 8)], sp_tl.at[s])
            for j in range(8):
                pltpu.sync_copy(sp_tl.at[s, j], trow_v.at[pl.ds(0, TAILN)])
                pltpu.sync_copy(
                    trow_v,
                    flat_hbm.at[pl.ds((c * CH + ch8 + j) * SEG + TAIL0, 40)],
                )

        return 0

    lax.fori_loop(0, NUM_COLS, col_body, 0)


def _gather_body(xf_hbm, flat_hbm, out_hbm, idx_v, g_v, sem):
    c0 = lax.axis_index("c")
    s = lax.axis_index("s")
    ch = c0 * NS + s

    def col_body(c, _):
        base = c * CH * SEG + ch * SEG + 1

        def q_body(q, _):
            pltpu.sync_copy(
                xf_hbm.at[pl.ds(c * BATCH + q * BQ, BQ)], idx_v
            )

            def s_body(i, _):
                off = i * 16
                idx_v[pl.ds(off, 16)] = idx_v[pl.ds(off, 16)] + base
                return 0

            lax.fori_loop(0, BQ // 16, s_body, 0)
            pltpu.async_copy(flat_hbm.at[idx_v], g_v, sem).wait()
            pltpu.sync_copy(g_v, out_hbm.at[c, ch, q])
            return 0

        lax.fori_loop(0, NBQ, q_body, 0)
        return 0

    lax.fori_loop(0, NUM_COLS, col_body, 0)


def kernel(x, tables):
    xT = x.T.reshape(N)
    tT = tables.transpose(0, 2, 1)
    mesh = plsc.VectorSubcoreMesh(core_axis_name="c", subcore_axis_name="s")

    tails = tT[:, :, TAIL0:]
    detile = pl.kernel(
        _detile_body,
        out_type=jax.ShapeDtypeStruct((FLAT,), jnp.float32),
        mesh=mesh,
        scratch_types=[
            pltpu.VMEM((SB,), jnp.float32),
            pltpu.VMEM((40,), jnp.float32),
            pltpu.VMEM_SHARED((NS, 8, SB), jnp.float32),
            pltpu.VMEM_SHARED((NS, 8, TAILN), jnp.float32),
        ],
        compiler_params=pltpu.CompilerParams(use_tc_tiling_on_sc=True),
    )
    flat = detile(tT, tails)

    gather = pl.kernel(
        _gather_body,
        out_type=jax.ShapeDtypeStruct((NUM_COLS, CH, NBQ, BQ), jnp.float32),
        mesh=mesh,
        scratch_types=[
            pltpu.VMEM((BQ,), jnp.int32),
            pltpu.VMEM((BQ,), jnp.float32),
            pltpu.SemaphoreType.DMA,
        ],
        compiler_params=pltpu.CompilerParams(use_tc_tiling_on_sc=False),
    )
    out = gather(xT, flat)
    return out.reshape(NUM_COLS, CH, BATCH).transpose(2, 0, 1)


# speedup vs baseline: 2.1360x; 2.1360x over previous
import functools

import jax
import jax.numpy as jnp
from jax import lax
from jax.experimental import pallas as pl
from jax.experimental.pallas import tpu as pltpu
from jax.experimental.pallas import tpu_sc as plsc

BATCH = 16384
NUM_COLS = 26
ROWS = 100001
CH = 32

N = BATCH * NUM_COLS
NC = 2
NS = 16
GROUPS_PER_CORE = 2
SEG = 100008
FLAT = NUM_COLS * CH * SEG
BQ = 4096
NBQ = BATCH // BQ


RB = 12544
RB_LAST = 12160
SB = 3072
SUBS = (SB, SB, SB, SB, 256)
SUBS_LAST = (SB, SB, SB, 2944)
TAIL0 = 99968
TAILN = ROWS - TAIL0


def _detile_body(
    tab_hbm, tails_hbm, flat_hbm, row_v, trow_v, sp_blk, sp_tl, esem, wsem
):
    c0 = lax.axis_index("c")
    s = lax.axis_index("s")
    grp = s // 8
    k = s % 8
    grp = c0 * GROUPS_PER_CORE + grp
    ch8 = grp * 8
    r0 = k * RB

    def col_body(c, _):
        def do_range(subs):
            off = 0
            for ext in subs:
                pltpu.sync_copy(
                    tab_hbm.at[c, pl.ds(ch8, 8), pl.ds(r0 + off, ext)],
                    sp_blk.at[s, :, pl.ds(0, ext)],
                )
                cps = [
                    pltpu.async_copy(
                        sp_blk.at[s, j, pl.ds(0, ext)],
                        row_v.at[pl.ds(j * SB, ext)],
                        esem,
                    )
                    for j in range(8)
                ]
                for cp in cps:
                    cp.wait()
                cps = [
                    pltpu.async_copy(
                        row_v.at[pl.ds(j * SB, ext)],
                        flat_hbm.at[
                            pl.ds((c * CH + ch8 + j) * SEG + r0 + off, ext)
                        ],
                        wsem,
                    )
                    for j in range(8)
                ]
                for cp in cps:
                    cp.wait()
                off += ext

        @pl.when(k < 7)
        def _main():
            do_range(SUBS)

        @pl.when(k == 7)
        def _last():
            do_range(SUBS_LAST)
            pltpu.sync_copy(tails_hbm.at[c, pl.ds(ch8, 8)], sp_tl.at[s])
            for j in range(8):
                pltpu.sync_copy(sp_tl.at[s, j], trow_v.at[pl.ds(0, TAILN)])
                pltpu.sync_copy(
                    trow_v,
                    flat_hbm.at[pl.ds((c * CH + ch8 + j) * SEG + TAIL0, 40)],
                )

        return 0

    lax.fori_loop(0, NUM_COLS, col_body, 0)


TOT = NUM_COLS * NBQ


def _gather_body(xf_hbm, flat_hbm, out_hbm, i0, i1, g0, g1, s0, s1):
    c0 = lax.axis_index("c")
    s = lax.axis_index("s")
    ch = c0 * NS + s

    def load_idx(t, iv):
        pltpu.sync_copy(xf_hbm.at[pl.ds(t * BQ, BQ)], iv)
        base = (t // NBQ) * (CH * SEG) + ch * SEG + 1

        def s_body(i, _):
            off = i * 16
            iv[pl.ds(off, 16)] = iv[pl.ds(off, 16)] + base
            return 0

        lax.fori_loop(0, BQ // 16, s_body, 0)

    def out_write(t, gv):
        pltpu.sync_copy(gv, out_hbm.at[t // NBQ, ch, t % NBQ])

    load_idx(0, i0)
    pltpu.async_copy(flat_hbm.at[i0], g0, s0)

    def pipe(it, _):
        t0 = 2 * it
        load_idx(t0 + 1, i1)
        pltpu.async_copy(flat_hbm.at[i1], g1, s1)
        pltpu.make_async_copy(flat_hbm.at[i0], g0, s0).wait()
        out_write(t0, g0)

        @pl.when(it + 1 < TOT // 2)
        def _next():
            load_idx(t0 + 2, i0)
            pltpu.async_copy(flat_hbm.at[i0], g0, s0)

        pltpu.make_async_copy(flat_hbm.at[i1], g1, s1).wait()
        out_write(t0 + 1, g1)
        return 0

    lax.fori_loop(0, TOT // 2, pipe, 0)


def kernel(x, tables):
    xT = x.T.reshape(N)
    tT = tables.transpose(0, 2, 1)
    mesh = plsc.VectorSubcoreMesh(core_axis_name="c", subcore_axis_name="s")

    tails = tT[:, :, TAIL0:]
    detile = pl.kernel(
        _detile_body,
        out_type=jax.ShapeDtypeStruct((FLAT,), jnp.float32),
        mesh=mesh,
        scratch_types=[
            pltpu.VMEM((8 * SB,), jnp.float32),
            pltpu.VMEM((40,), jnp.float32),
            pltpu.VMEM_SHARED((NS, 8, SB), jnp.float32),
            pltpu.VMEM_SHARED((NS, 8, TAILN), jnp.float32),
            pltpu.SemaphoreType.DMA,
            pltpu.SemaphoreType.DMA,
        ],
        compiler_params=pltpu.CompilerParams(use_tc_tiling_on_sc=True),
    )
    flat = detile(tT, tails)

    gather = pl.kernel(
        _gather_body,
        out_type=jax.ShapeDtypeStruct((NUM_COLS, CH, NBQ, BQ), jnp.float32),
        mesh=mesh,
        scratch_types=[
            pltpu.VMEM((BQ,), jnp.int32),
            pltpu.VMEM((BQ,), jnp.int32),
            pltpu.VMEM((BQ,), jnp.float32),
            pltpu.VMEM((BQ,), jnp.float32),
            pltpu.SemaphoreType.DMA,
            pltpu.SemaphoreType.DMA,
        ],
        compiler_params=pltpu.CompilerParams(use_tc_tiling_on_sc=False),
    )
    out = gather(xT, flat)
    return out.reshape(NUM_COLS, CH, BATCH).transpose(2, 0, 1)


# speedup vs baseline: 2.1676x; 1.0148x over previous
import functools

import jax
import jax.numpy as jnp
from jax import lax
from jax.experimental import pallas as pl
from jax.experimental.pallas import tpu as pltpu
from jax.experimental.pallas import tpu_sc as plsc

BATCH = 16384
NUM_COLS = 26
ROWS = 100001
CH = 32

N = BATCH * NUM_COLS
NC = 2
NS = 16
GROUPS_PER_CORE = 2
SEG = 100008
FLAT = NUM_COLS * CH * SEG
BQ = 8192
NBQ = BATCH // BQ


RB = 12544
RB_LAST = 12160
SB = 3072
SUBS = (SB, SB, SB, SB, 256)
SUBS_LAST = (SB, SB, SB, 2944)
TAIL0 = 99968
TAILN = ROWS - TAIL0


def _detile_body(
    tab_hbm, tails_hbm, flat_hbm, row_v, trow_v, sp_blk, sp_tl, esem, wsem
):
    c0 = lax.axis_index("c")
    s = lax.axis_index("s")
    grp = s // 8
    k = s % 8
    grp = c0 * GROUPS_PER_CORE + grp
    ch8 = grp * 8
    r0 = k * RB

    def col_body(c, _):
        def do_range(subs):
            off = 0
            for ext in subs:
                pltpu.sync_copy(
                    tab_hbm.at[c, pl.ds(ch8, 8), pl.ds(r0 + off, ext)],
                    sp_blk.at[s, :, pl.ds(0, ext)],
                )
                cps = [
                    pltpu.async_copy(
                        sp_blk.at[s, j, pl.ds(0, ext)],
                        row_v.at[pl.ds(j * SB, ext)],
                        esem,
                    )
                    for j in range(8)
                ]
                for cp in cps:
                    cp.wait()
                cps = [
                    pltpu.async_copy(
                        row_v.at[pl.ds(j * SB, ext)],
                        flat_hbm.at[
                            pl.ds((c * CH + ch8 + j) * SEG + r0 + off, ext)
                        ],
                        wsem,
                    )
                    for j in range(8)
                ]
                for cp in cps:
                    cp.wait()
                off += ext

        @pl.when(k < 7)
        def _main():
            do_range(SUBS)

        @pl.when(k == 7)
        def _last():
            do_range(SUBS_LAST)
            pltpu.sync_copy(tails_hbm.at[c, pl.ds(ch8, 8)], sp_tl.at[s])
            for j in range(8):
                pltpu.sync_copy(sp_tl.at[s, j], trow_v.at[pl.ds(0, TAILN)])
                pltpu.sync_copy(
                    trow_v,
                    flat_hbm.at[pl.ds((c * CH + ch8 + j) * SEG + TAIL0, 40)],
                )

        return 0

    lax.fori_loop(0, NUM_COLS, col_body, 0)


TOT = NUM_COLS * NBQ


def _gather_body(xf_hbm, flat_hbm, out_hbm, i0, i1, g0, g1, s0, s1):
    c0 = lax.axis_index("c")
    s = lax.axis_index("s")
    ch = c0 * NS + s

    def load_idx(t, iv):
        pltpu.sync_copy(xf_hbm.at[pl.ds(t * BQ, BQ)], iv)
        base = (t // NBQ) * (CH * SEG) + ch * SEG + 1

        def s_body(i, _):
            off = i * 16
            iv[pl.ds(off, 16)] = iv[pl.ds(off, 16)] + base
            return 0

        lax.fori_loop(0, BQ // 16, s_body, 0)

    def out_write(t, gv):
        pltpu.sync_copy(gv, out_hbm.at[t // NBQ, ch, t % NBQ])

    load_idx(0, i0)
    pltpu.async_copy(flat_hbm.at[i0], g0, s0)

    def pipe(it, _):
        t0 = 2 * it
        load_idx(t0 + 1, i1)
        pltpu.async_copy(flat_hbm.at[i1], g1, s1)
        pltpu.make_async_copy(flat_hbm.at[i0], g0, s0).wait()
        out_write(t0, g0)

        @pl.when(it + 1 < TOT // 2)
        def _next():
            load_idx(t0 + 2, i0)
            pltpu.async_copy(flat_hbm.at[i0], g0, s0)

        pltpu.make_async_copy(flat_hbm.at[i1], g1, s1).wait()
        out_write(t0 + 1, g1)
        return 0

    lax.fori_loop(0, TOT // 2, pipe, 0)


def kernel(x, tables):
    xT = x.T.reshape(N)
    tT = tables.transpose(0, 2, 1)
    mesh = plsc.VectorSubcoreMesh(core_axis_name="c", subcore_axis_name="s")

    tails = tT[:, :, TAIL0:]
    detile = pl.kernel(
        _detile_body,
        out_type=jax.ShapeDtypeStruct((FLAT,), jnp.float32),
        mesh=mesh,
        scratch_types=[
            pltpu.VMEM((8 * SB,), jnp.float32),
            pltpu.VMEM((40,), jnp.float32),
            pltpu.VMEM_SHARED((NS, 8, SB), jnp.float32),
            pltpu.VMEM_SHARED((NS, 8, TAILN), jnp.float32),
            pltpu.SemaphoreType.DMA,
            pltpu.SemaphoreType.DMA,
        ],
        compiler_params=pltpu.CompilerParams(use_tc_tiling_on_sc=True),
    )
    flat = detile(tT, tails)

    gather = pl.kernel(
        _gather_body,
        out_type=jax.ShapeDtypeStruct((NUM_COLS, CH, NBQ, BQ), jnp.float32),
        mesh=mesh,
        scratch_types=[
            pltpu.VMEM((BQ,), jnp.int32),
            pltpu.VMEM((BQ,), jnp.int32),
            pltpu.VMEM((BQ,), jnp.float32),
            pltpu.VMEM((BQ,), jnp.float32),
            pltpu.SemaphoreType.DMA,
            pltpu.SemaphoreType.DMA,
        ],
        compiler_params=pltpu.CompilerParams(use_tc_tiling_on_sc=False),
    )
    out = gather(xT, flat)
    return out.reshape(NUM_COLS, CH, BATCH).transpose(2, 0, 1)
